# Initial kernel scaffold; baseline (speedup 1.0000x reference)
import functools

import jax
import jax.numpy as jnp
from jax import lax
from jax.experimental import pallas as pl
from jax.experimental.pallas import tpu as pltpu
from jax.experimental.pallas import tpu_sc as plsc

BIG = jnp.float32(1e9)
K = 16
N = 4096
NCHUNK = N // 16


def _sort16(x, descending=False):
    k, _ = plsc.sort_key_val(x, x, descending=descending)
    return k


def _merge_top(u, l, xd):
    lo1 = jnp.minimum(u, xd)
    u2 = _sort16(jnp.maximum(u, xd), descending=False)
    hi2 = jnp.maximum(l, _sort16(lo1, descending=True))
    l2 = _sort16(hi2, descending=False)
    return u2, l2


def _valid_mask(e, st):
    inf = jnp.float32(jnp.inf)
    fin = (jnp.abs(e) != inf) & (jnp.abs(st) != inf)
    nz = (e * st) != jnp.float32(0.0)
    df = (e - st) != jnp.float32(0.0)
    return jnp.where(fin & nz & df, jnp.float32(1.0), jnp.float32(0.0))


def _sqrt16(xv):
    bits = plsc.bitcast(xv, jnp.int32)
    y = plsc.bitcast((bits >> 1) + jnp.int32(0x1FBD1DF5), jnp.float32)
    half = jnp.float32(0.5)
    for _ in range(4):
        y = half * (y + xv / y)
    return y


def _toploss_body(imgs_hbm, out_hbm, img_v, stage_v, s32_v, d_v, g_v, sc_v,
                  res_v, diag_sh, loss_sh):
    c = lax.axis_index("c")
    s = lax.axis_index("s")
    iota = lax.iota(jnp.int32, 16)

    @pl.when(s < 12)
    def _phase1():
        row = 6 * c + s + jnp.where(s < 6, 0, 6)
        pltpu.sync_copy(imgs_hbm.at[pl.ds(row * N, N)], img_v)

        x0 = img_v[pl.ds(0, 16)]
        x1 = img_v[pl.ds(16, 16)]
        a = _sort16(x0, descending=False)
        b = _sort16(x1, descending=True)
        u = _sort16(jnp.maximum(a, b), descending=False)
        l = _sort16(jnp.minimum(a, b), descending=False)
        an = _sort16(-x0, descending=False)
        bn = _sort16(-x1, descending=True)
        bu = _sort16(jnp.maximum(an, bn), descending=False)
        bl = _sort16(jnp.minimum(an, bn), descending=False)

        def body(k, carry):
            u, l, bu, bl = carry
            x = img_v[pl.ds(k * 16, 16)]
            u, l = _merge_top(u, l, _sort16(x, descending=True))
            bu, bl = _merge_top(bu, bl, _sort16(-x, descending=True))
            return u, l, bu, bl

        u, l, bu, bl = lax.fori_loop(2, NCHUNK, body, (u, l, bu, bl))

        s32_v[pl.ds(0, 16)] = _sort16(u, descending=True)
        s32_v[pl.ds(16, 16)] = _sort16(l, descending=True)
        stage_v[pl.ds(0, 16)] = plsc.load_gather(s32_v, [2 * iota])
        stage_v[pl.ds(16, 16)] = plsc.load_gather(s32_v, [2 * iota + 1])
        s32_v[pl.ds(0, 16)] = -_sort16(bu, descending=True)
        s32_v[pl.ds(16, 16)] = -_sort16(bl, descending=True)
        stage_v[pl.ds(32, 16)] = plsc.load_gather(s32_v, [2 * iota + 1])
        stage_v[pl.ds(48, 16)] = plsc.load_gather(s32_v, [2 * iota])
        pltpu.sync_copy(stage_v, diag_sh.at[s])

    plsc.subcore_barrier()

    @pl.when(s < 12)
    def _phase2():
        q = s // 6
        sig = s - 6 * q
        pltpu.sync_copy(diag_sh.at[sig], d_v)
        pltpu.sync_copy(diag_sh.at[6 + sig], g_v)
        q32 = q * 32
        de = d_v[pl.ds(q32, 16)]
        dst = d_v[pl.ds(q32 + 16, 16)]
        ge = g_v[pl.ds(q32, 16)]
        gs = g_v[pl.ds(q32 + 16, 16)]

        m = _valid_mask(de, dst)
        mg = _valid_mask(ge, gs)
        pen = (jnp.float32(1.0) - mg) * BIG

        sc_v[pl.ds(0, 16)] = de
        sc_v[pl.ds(16, 16)] = dst
        sc_v[pl.ds(32, 16)] = m

        used = jnp.zeros((16,), jnp.float32)
        acc = jnp.float32(0.0)
        one = jnp.float32(1.0)
        for i in range(K):
            e_i = sc_v[i]
            s_i = sc_v[16 + i]
            m_i = sc_v[32 + i]
            dx = e_i - ge
            dy = s_i - gs
            crow = dx * dx + dy * dy + pen + used * BIG
            mn = jnp.min(crow)
            j = plsc.all_reduce_ffs(crow == mn)
            oh = iota == j
            mg_j = jnp.sum(jnp.where(oh, mg, jnp.float32(0.0)))
            ge_j = jnp.sum(jnp.where(oh, ge, jnp.float32(0.0)))
            gs_j = jnp.sum(jnp.where(oh, gs, jnp.float32(0.0)))
            take = m_i * mg_j
            rm = (e_i + s_i) * jnp.float32(0.5)
            o_e = take * ge_j + (one - take) * rm
            o_s = take * gs_j + (one - take) * rm
            dd_e = (e_i - o_e) * m_i
            dd_s = (s_i - o_s) * m_i
            acc = acc + dd_e * dd_e + dd_s * dd_s
            used = used + jnp.where(oh, take, jnp.float32(0.0))

        xv = acc + jnp.float32(1e-12) + jnp.zeros((16,), jnp.float32)
        res_v[...] = _sqrt16(xv)
        pltpu.sync_copy(res_v, loss_sh.at[s])

    plsc.subcore_barrier()

    @pl.when(s == 0)
    def _phase3():
        total = jnp.zeros((16,), jnp.float32)
        for w in range(12):
            pltpu.sync_copy(loss_sh.at[w], res_v)
            total = total + res_v[...]
        res_v[...] = total * jnp.float32(1.0 / 12.0)
        pltpu.sync_copy(res_v, out_hbm.at[pl.ds(c * 16, 16)])


@functools.partial(
    pl.kernel,
    out_type=jax.ShapeDtypeStruct((32,), jnp.float32),
    mesh=plsc.VectorSubcoreMesh(core_axis_name="c", subcore_axis_name="s"),
    scratch_types=[
        pltpu.VMEM((N,), jnp.float32),
        pltpu.VMEM((64,), jnp.float32),
        pltpu.VMEM((32,), jnp.float32),
        pltpu.VMEM((64,), jnp.float32),
        pltpu.VMEM((64,), jnp.float32),
        pltpu.VMEM((48,), jnp.float32),
        pltpu.VMEM((16,), jnp.float32),
        pltpu.VMEM_SHARED((12, 64), jnp.float32),
        pltpu.VMEM_SHARED((12, 16), jnp.float32),
    ],
)
def _toploss(imgs_hbm, out_hbm, img_v, stage_v, s32_v, d_v, g_v, sc_v, res_v,
             diag_sh, loss_sh):
    _toploss_body(imgs_hbm, out_hbm, img_v, stage_v, s32_v, d_v, g_v, sc_v,
                  res_v, diag_sh, loss_sh)


@jax.jit
def kernel(beta, ground):
    imgs = jnp.concatenate(
        [beta.reshape(12, N), ground.reshape(12, N)], axis=0
    ).reshape(-1)
    out = _toploss(imgs)
    return out[0] + out[16]


# speedup vs baseline: 63.9816x; 63.9816x over previous
import functools

import jax
import jax.numpy as jnp
import numpy as np
from jax import lax
from jax.experimental import pallas as pl
from jax.experimental.pallas import tpu as pltpu
from jax.experimental.pallas import tpu_sc as plsc

BIG = np.float32(1e9)
K = 16
N = 4096
NCHUNK = N // 16


def _sort16(x, descending=False):
    k, _ = plsc.sort_key_val(x, x, descending=descending)
    return k


def _merge_top(u, l, xd):
    lo1 = jnp.minimum(u, xd)
    u2 = _sort16(jnp.maximum(u, xd), descending=False)
    hi2 = jnp.maximum(l, _sort16(lo1, descending=True))
    l2 = _sort16(hi2, descending=False)
    return u2, l2


def _valid_mask(e, st):
    inf = np.float32(np.inf)
    fin = (jnp.abs(e) != inf) & (jnp.abs(st) != inf)
    nz = (e * st) != np.float32(0.0)
    df = (e - st) != np.float32(0.0)
    return jnp.where(fin & nz & df, np.float32(1.0), np.float32(0.0))


def _sqrt16(xv):
    bits = plsc.bitcast(xv, jnp.int32)
    y = plsc.bitcast((bits >> 1) + np.int32(0x1FBD1DF5), jnp.float32)
    half = np.float32(0.5)
    for _ in range(4):
        y = half * (y + xv / y)
    return y


def _toploss_body(imgs_hbm, out_hbm, diag_hbm, loss_hbm, img_v, stage_v,
                  s32_v, d_v, g_v, res_v):
    c = lax.axis_index("c")
    s = lax.axis_index("s")
    iota = lax.iota(jnp.int32, 16)

    @pl.when(s < 12)
    def _phase1():
        row = 6 * c + s + jnp.where(s < 6, 0, 6)
        pltpu.sync_copy(imgs_hbm.at[pl.ds(row * N, N)], img_v)

        x0 = img_v[pl.ds(0, 16)]
        x1 = img_v[pl.ds(16, 16)]
        a = _sort16(x0, descending=False)
        b = _sort16(x1, descending=True)
        u = _sort16(jnp.maximum(a, b), descending=False)
        l = _sort16(jnp.minimum(a, b), descending=False)
        an = _sort16(-x0, descending=False)
        bn = _sort16(-x1, descending=True)
        bu = _sort16(jnp.maximum(an, bn), descending=False)
        bl = _sort16(jnp.minimum(an, bn), descending=False)

        def body(k, carry):
            u, l, bu, bl = carry
            x = img_v[pl.ds(k * 16, 16)]
            u, l = _merge_top(u, l, _sort16(x, descending=True))
            bu, bl = _merge_top(bu, bl, _sort16(-x, descending=True))
            return u, l, bu, bl

        u, l, bu, bl = lax.fori_loop(2, NCHUNK, body, (u, l, bu, bl))

        s32_v[pl.ds(0, 16)] = _sort16(u, descending=True)
        s32_v[pl.ds(16, 16)] = _sort16(l, descending=True)
        stage_v[pl.ds(0, 16)] = plsc.load_gather(s32_v, [2 * iota])
        stage_v[pl.ds(16, 16)] = plsc.load_gather(s32_v, [2 * iota + 1])
        s32_v[pl.ds(0, 16)] = -_sort16(bu, descending=True)
        s32_v[pl.ds(16, 16)] = -_sort16(bl, descending=True)
        stage_v[pl.ds(32, 16)] = plsc.load_gather(s32_v, [2 * iota + 1])
        stage_v[pl.ds(48, 16)] = plsc.load_gather(s32_v, [2 * iota])
        pltpu.sync_copy(stage_v, diag_hbm.at[pl.ds((12 * c + s) * 64, 64)])

    plsc.subcore_barrier()

    @pl.when(s < 12)
    def _phase2():
        q = s // 6
        sig = s - 6 * q
        pltpu.sync_copy(diag_hbm.at[pl.ds((12 * c + sig) * 64, 64)], d_v)
        pltpu.sync_copy(diag_hbm.at[pl.ds((12 * c + 6 + sig) * 64, 64)], g_v)
        q32 = q * 32
        de = d_v[pl.ds(q32, 16)]
        dst = d_v[pl.ds(q32 + 16, 16)]
        ge = g_v[pl.ds(q32, 16)]
        gs = g_v[pl.ds(q32 + 16, 16)]

        m = _valid_mask(de, dst)
        mg = _valid_mask(ge, gs)
        pen = (np.float32(1.0) - mg) * BIG

        used = jnp.zeros((16,), jnp.float32)
        acc = np.float32(0.0)
        one = np.float32(1.0)
        for i in range(K):
            e_i = de[i]
            s_i = dst[i]
            m_i = m[i]
            dx = e_i - ge
            dy = s_i - gs
            crow = dx * dx + dy * dy + pen + used * BIG
            mn = jnp.min(crow)
            j = plsc.all_reduce_ffs(crow == mn)
            oh = iota == j
            mg_j = jnp.sum(jnp.where(oh, mg, np.float32(0.0)))
            ge_j = jnp.sum(jnp.where(oh, ge, np.float32(0.0)))
            gs_j = jnp.sum(jnp.where(oh, gs, np.float32(0.0)))
            take = m_i * mg_j
            rm = (e_i + s_i) * np.float32(0.5)
            o_e = take * ge_j + (one - take) * rm
            o_s = take * gs_j + (one - take) * rm
            dd_e = (e_i - o_e) * m_i
            dd_s = (s_i - o_s) * m_i
            acc = acc + dd_e * dd_e + dd_s * dd_s
            used = used + jnp.where(oh, take, np.float32(0.0))

        xv = acc + np.float32(1e-12) + jnp.zeros((16,), jnp.float32)
        res_v[...] = _sqrt16(xv)
        pltpu.sync_copy(res_v, loss_hbm.at[pl.ds((12 * c + s) * 16, 16)])

    plsc.subcore_barrier()

    @pl.when(s == 0)
    def _phase3():
        total = jnp.zeros((16,), jnp.float32)
        for w in range(12):
            pltpu.sync_copy(loss_hbm.at[pl.ds((12 * c + w) * 16, 16)], res_v)
            total = total + res_v[...]
        res_v[...] = total * np.float32(1.0 / 12.0)
        pltpu.sync_copy(res_v, out_hbm.at[pl.ds(c * 16, 16)])


@functools.partial(
    pl.kernel,
    out_type=(
        jax.ShapeDtypeStruct((32,), jnp.float32),
        jax.ShapeDtypeStruct((24 * 64,), jnp.float32),
        jax.ShapeDtypeStruct((24 * 16,), jnp.float32),
    ),
    mesh=plsc.VectorSubcoreMesh(core_axis_name="c", subcore_axis_name="s",
                                num_cores=2, num_subcores=16),
    compiler_params=pltpu.CompilerParams(needs_layout_passes=False),
    scratch_types=[
        pltpu.VMEM((N,), jnp.float32),
        pltpu.VMEM((64,), jnp.float32),
        pltpu.VMEM((32,), jnp.float32),
        pltpu.VMEM((64,), jnp.float32),
        pltpu.VMEM((64,), jnp.float32),
        pltpu.VMEM((16,), jnp.float32),
    ],
)
def _toploss(imgs_hbm, out_hbm, diag_hbm, loss_hbm, img_v, stage_v, s32_v,
             d_v, g_v, res_v):
    _toploss_body(imgs_hbm, out_hbm, diag_hbm, loss_hbm, img_v, stage_v,
                  s32_v, d_v, g_v, res_v)


@jax.jit
def kernel(beta, ground):
    imgs = jnp.concatenate(
        [beta.reshape(12, N), ground.reshape(12, N)], axis=0
    ).reshape(-1)
    out, _, _ = _toploss(imgs)
    return out[0] + out[16]
